# baseline (speedup 1.0000x reference)
import math

import numpy as np
import jax
import jax.numpy as jnp
from jax import lax
from jax.experimental import pallas as pl
from jax.experimental.pallas import tpu as pltpu

_B, _L, _D = 4, 8192, 768
_H, _DK, _U = 12, 64, 50
_NU = _H * _U
_BL = 512
_BK = 512

_F32 = jnp.float32
_BF16 = jnp.bfloat16

_IDX_SAMPLE = np.asarray(
    jax.random.randint(jax.random.key(1234), (_U,), 0, _L, dtype=jnp.int32)
)


def _mmb(a, b, dims):
    return lax.dot_general(a.astype(_BF16), b.astype(_BF16),
                           (dims, ((), ())), preferred_element_type=_F32)


def _mmf(a, b, dims):
    return lax.dot_general(a, b, (dims, ((), ())),
                           preferred_element_type=_F32,
                           precision=lax.Precision.HIGHEST)


def _p0_body(ksr_ref, wk_ref, bkc_ref, ks2_ref):
    kst = _mmb(wk_ref[...], ksr_ref[0], ((1,), (1,))) + bkc_ref[...]
    row = lax.broadcasted_iota(jnp.int32, (_D, _U), 0) // _DK
    parts = [jnp.where(row == h, kst, 0.0) for h in range(_H)]
    ks2_ref[0] = jnp.concatenate(parts, axis=1)


def _p0(ksamp_raw, Wk, bk_col):
    return pl.pallas_call(
        _p0_body,
        grid=(_B,),
        in_specs=[
            pl.BlockSpec((1, _U, _D), lambda b: (b, 0, 0)),
            pl.BlockSpec((_D, _D), lambda b: (0, 0)),
            pl.BlockSpec((_D, 1), lambda b: (0, 0)),
        ],
        out_specs=pl.BlockSpec((1, _D, _NU), lambda b: (b, 0, 0)),
        out_shape=jax.ShapeDtypeStruct((_B, _D, _NU), _F32),
    )(ksamp_raw, Wk, bk_col)


def _p1_body(q_ref, wq_ref, bq_ref, ks2_ref, m_ref):
    qh = _mmb(q_ref[0], wq_ref[...], ((1,), (1,))) + bq_ref[...]
    s = _mmb(qh, ks2_ref[0], ((1,), (0,)))
    for h in range(_H):
        sh = s[:, h * _U:(h + 1) * _U]
        m_ref[0, h, :] = jnp.max(sh, axis=1) - jnp.sum(sh, axis=1) / _U


def _p1(q, Wq, bq2, ks2):
    return pl.pallas_call(
        _p1_body,
        grid=(_B, _L // _BL),
        in_specs=[
            pl.BlockSpec((1, _BL, _D), lambda b, l: (b, l, 0)),
            pl.BlockSpec((_D, _D), lambda b, l: (0, 0)),
            pl.BlockSpec((1, _D), lambda b, l: (0, 0)),
            pl.BlockSpec((1, _D, _NU), lambda b, l: (b, 0, 0)),
        ],
        out_specs=pl.BlockSpec((1, _H, _BL), lambda b, l: (b, 0, l)),
        out_shape=jax.ShapeDtypeStruct((_B, _H, _L), _F32),
    )(q, Wq, bq2, ks2)


def _p3_body(qw_ref, wq_ref, bq_ref, qtm_ref):
    qt = _mmb(qw_ref[0], wq_ref[...], ((1,), (1,))) + bq_ref[...]
    rh = lax.broadcasted_iota(jnp.int32, (_NU, _D), 0) // _U
    ch = lax.broadcasted_iota(jnp.int32, (_NU, _D), 1) // _DK
    qtm_ref[0] = jnp.where(rh == ch, qt, 0.0)


def _p3(q_win, Wq, bq2):
    return pl.pallas_call(
        _p3_body,
        grid=(_B,),
        in_specs=[
            pl.BlockSpec((1, _NU, _D), lambda b: (b, 0, 0)),
            pl.BlockSpec((_D, _D), lambda b: (0, 0)),
            pl.BlockSpec((1, _D), lambda b: (0, 0)),
        ],
        out_specs=pl.BlockSpec((1, _NU, _D), lambda b: (b, 0, 0)),
        out_shape=jax.ShapeDtypeStruct((_B, _NU, _D), _F32),
    )(q_win, Wq, bq2)


def _p4_body(qtm_ref, k_ref, v_ref, wk_ref, bk_ref, wv_ref, bv_ref,
             ov_ref, vs_ref, acc_ref, m_ref, l_ref, vsum_ref):
    kb = pl.program_id(1)

    @pl.when(kb == 0)
    def _init():
        acc_ref[...] = jnp.zeros_like(acc_ref)
        m_ref[...] = jnp.full_like(m_ref, -3e38)
        l_ref[...] = jnp.zeros_like(l_ref)
        vsum_ref[...] = jnp.zeros_like(vsum_ref)

    kh = _mmb(k_ref[0], wk_ref[...], ((1,), (1,))) + bk_ref[...]
    vh = _mmb(v_ref[0], wv_ref[...], ((1,), (1,))) + bv_ref[...]

    lg = _mmb(qtm_ref[0], kh, ((1,), (1,))) * 0.125
    m_old = m_ref[...]
    m_new = jnp.maximum(m_old, jnp.max(lg, axis=1, keepdims=True))
    alpha = jnp.exp(m_old - m_new)
    p = jnp.exp(lg - m_new)
    l_ref[...] = l_ref[...] * alpha + jnp.sum(p, axis=1, keepdims=True)
    m_ref[...] = m_new
    acc_ref[...] = acc_ref[...] * alpha + _mmb(p, vh, ((1,), (0,)))
    vsum_ref[0:1, :] = vsum_ref[0:1, :] + jnp.sum(vh, axis=0, keepdims=True)

    @pl.when(kb == (_L // _BK) - 1)
    def _fin():
        ov_ref[0] = acc_ref[...] / l_ref[...]
        vs_ref[0] = vsum_ref[...]


def _p4(qtm, k, v, Wk, bk2, Wv, bv2):
    return pl.pallas_call(
        _p4_body,
        grid=(_B, _L // _BK),
        in_specs=[
            pl.BlockSpec((1, _NU, _D), lambda b, kb: (b, 0, 0)),
            pl.BlockSpec((1, _BK, _D), lambda b, kb: (b, kb, 0)),
            pl.BlockSpec((1, _BK, _D), lambda b, kb: (b, kb, 0)),
            pl.BlockSpec((_D, _D), lambda b, kb: (0, 0)),
            pl.BlockSpec((1, _D), lambda b, kb: (0, 0)),
            pl.BlockSpec((_D, _D), lambda b, kb: (0, 0)),
            pl.BlockSpec((1, _D), lambda b, kb: (0, 0)),
        ],
        out_specs=[
            pl.BlockSpec((1, _NU, _D), lambda b, kb: (b, 0, 0)),
            pl.BlockSpec((1, 8, _D), lambda b, kb: (b, 0, 0)),
        ],
        out_shape=[
            jax.ShapeDtypeStruct((_B, _NU, _D), _F32),
            jax.ShapeDtypeStruct((_B, 8, _D), _F32),
        ],
        scratch_shapes=[
            pltpu.VMEM((_NU, _D), _F32),
            pltpu.VMEM((_NU, 1), _F32),
            pltpu.VMEM((_NU, 1), _F32),
            pltpu.VMEM((8, _D), _F32),
        ],
    )(qtm, k, v, Wk, bk2, Wv, bv2)


def _p6_body(ov_ref, vs_ref, wo_ref, bo_ref, delta_ref, base_ref):
    vmean = vs_ref[0, 0:1, :] * (1.0 / _L)
    vmb = vmean.astype(_BF16).astype(_F32)
    ov = ov_ref[0]
    parts = []
    for h in range(_H):
        c0 = h * _DK
        ot = ov[h * _U:(h + 1) * _U, c0:c0 + _DK]
        d_h = ot.astype(_BF16).astype(_F32) - vmb[0:1, c0:c0 + _DK]
        wo_h = wo_ref[:, c0:c0 + _DK].astype(_BF16).astype(_F32)
        parts.append(_mmf(d_h, wo_h, ((1,), (1,))))
    delta_ref[0] = jnp.concatenate(parts, axis=0)
    base_ref[0] = _mmb(vmean, wo_ref[...], ((1,), (1,))) + bo_ref[...]


def _p6(out_v, vs, Wo, bo2):
    return pl.pallas_call(
        _p6_body,
        grid=(_B,),
        in_specs=[
            pl.BlockSpec((1, _NU, _D), lambda b: (b, 0, 0)),
            pl.BlockSpec((1, 8, _D), lambda b: (b, 0, 0)),
            pl.BlockSpec((_D, _D), lambda b: (0, 0)),
            pl.BlockSpec((1, _D), lambda b: (0, 0)),
        ],
        out_specs=[
            pl.BlockSpec((1, _NU, _D), lambda b: (b, 0, 0)),
            pl.BlockSpec((1, 1, _D), lambda b: (b, 0, 0)),
        ],
        out_shape=[
            jax.ShapeDtypeStruct((_B, _NU, _D), _F32),
            jax.ShapeDtypeStruct((_B, 1, _D), _F32),
        ],
    )(out_v, vs, Wo, bo2)


def kernel(q, k, v, Wq, bq, Wk, bk, Wv, bv, Wo, bo):
    bq2 = bq.reshape(1, _D)
    bk2 = bk.reshape(1, _D)
    bv2 = bv.reshape(1, _D)
    bo2 = bo.reshape(1, _D)
    bk_col = bk.reshape(_D, 1)

    ksamp_raw = k[:, _IDX_SAMPLE, :]
    ks2 = _p0(ksamp_raw, Wk, bk_col)
    m = _p1(q, Wq, bq2, ks2)

    _, top_idx = lax.top_k(m, _U)
    ti_flat = top_idx.reshape(_B, _NU)

    q_win = jnp.take_along_axis(q, ti_flat[..., None], axis=1)
    qtm = _p3(q_win, Wq, bq2)
    out_v, vs = _p4(qtm, k, v, Wk, bk2, Wv, bv2)
    delta, base = _p6(out_v, vs, Wo, bo2)

    out = jnp.broadcast_to(base, (_B, _L, _D))
    b_idx = jnp.arange(_B)[:, None]
    out = out.at[b_idx, ti_flat].add(delta)
    return out


# speedup vs baseline: 1.0542x; 1.0542x over previous
import math

import numpy as np
import jax
import jax.numpy as jnp
from jax import lax
from jax.experimental import pallas as pl
from jax.experimental.pallas import tpu as pltpu

_B, _L, _D = 4, 8192, 768
_H, _DK, _U = 12, 64, 50
_NU = _H * _U
_BL = 512
_BK = 512

_F32 = jnp.float32
_BF16 = jnp.bfloat16

_IDX_SAMPLE = np.asarray([
    2357, 5665, 1885, 32, 4380, 7910, 4774, 7902, 3521, 7587, 3552, 5218,
    5424, 4781, 7884, 124, 1912, 5929, 235, 7940, 3146, 7584, 5586, 5739,
    6092, 5525, 4343, 2866, 7935, 7898, 6327, 7445, 7905, 2412, 3345, 1560,
    170, 4466, 7592, 7928, 2565, 5869, 5844, 1295, 2306, 1174, 5180, 412,
    3021, 7932], dtype=np.int32)


def _mmb(a, b, dims):
    return lax.dot_general(a.astype(_BF16), b.astype(_BF16),
                           (dims, ((), ())), preferred_element_type=_F32)


def _mmf(a, b, dims):
    return lax.dot_general(a, b, (dims, ((), ())),
                           preferred_element_type=_F32,
                           precision=lax.Precision.HIGHEST)


def _p0_body(ksr_ref, wk_ref, bkc_ref, ks2_ref):
    kst = _mmb(wk_ref[...], ksr_ref[0], ((1,), (1,))) + bkc_ref[...]
    row = lax.broadcasted_iota(jnp.int32, (_D, _U), 0) // _DK
    parts = [jnp.where(row == h, kst, 0.0) for h in range(_H)]
    ks2_ref[0] = jnp.concatenate(parts, axis=1)


def _p0(ksamp_raw, Wk, bk_col):
    return pl.pallas_call(
        _p0_body,
        grid=(_B,),
        in_specs=[
            pl.BlockSpec((1, _U, _D), lambda b: (b, 0, 0)),
            pl.BlockSpec((_D, _D), lambda b: (0, 0)),
            pl.BlockSpec((_D, 1), lambda b: (0, 0)),
        ],
        out_specs=pl.BlockSpec((1, _D, _NU), lambda b: (b, 0, 0)),
        out_shape=jax.ShapeDtypeStruct((_B, _D, _NU), _F32),
    )(ksamp_raw, Wk, bk_col)


def _p1_body(q_ref, wq_ref, bq_ref, ks2_ref, m_ref):
    qh = _mmb(q_ref[0], wq_ref[...], ((1,), (1,))) + bq_ref[...]
    s = _mmb(qh, ks2_ref[0], ((1,), (0,)))
    for h in range(_H):
        sh = s[:, h * _U:(h + 1) * _U]
        m_ref[0, h, :] = jnp.max(sh, axis=1) - jnp.sum(sh, axis=1) / _U


def _p1(q, Wq, bq2, ks2):
    return pl.pallas_call(
        _p1_body,
        grid=(_B, _L // _BL),
        in_specs=[
            pl.BlockSpec((1, _BL, _D), lambda b, l: (b, l, 0)),
            pl.BlockSpec((_D, _D), lambda b, l: (0, 0)),
            pl.BlockSpec((1, _D), lambda b, l: (0, 0)),
            pl.BlockSpec((1, _D, _NU), lambda b, l: (b, 0, 0)),
        ],
        out_specs=pl.BlockSpec((1, _H, _BL), lambda b, l: (b, 0, l)),
        out_shape=jax.ShapeDtypeStruct((_B, _H, _L), _F32),
    )(q, Wq, bq2, ks2)


def _p3_body(qw_ref, wq_ref, bq_ref, wk_ref, qe_ref):
    qt = _mmb(qw_ref[0], wq_ref[...], ((1,), (1,))) + bq_ref[...]
    rh = lax.broadcasted_iota(jnp.int32, (_NU, _D), 0) // _U
    ch = lax.broadcasted_iota(jnp.int32, (_NU, _D), 1) // _DK
    qtm = jnp.where(rh == ch, qt, 0.0)
    qe_ref[0] = _mmb(qtm, wk_ref[...], ((1,), (0,))) * 0.125


def _p3(q_win, Wq, bq2, Wk):
    return pl.pallas_call(
        _p3_body,
        grid=(_B,),
        in_specs=[
            pl.BlockSpec((1, _NU, _D), lambda b: (b, 0, 0)),
            pl.BlockSpec((_D, _D), lambda b: (0, 0)),
            pl.BlockSpec((1, _D), lambda b: (0, 0)),
            pl.BlockSpec((_D, _D), lambda b: (0, 0)),
        ],
        out_specs=pl.BlockSpec((1, _NU, _D), lambda b: (b, 0, 0)),
        out_shape=jax.ShapeDtypeStruct((_B, _NU, _D), _F32),
    )(q_win, Wq, bq2, Wk)


def _p4_body(qe_ref, k_ref, v_ref, ov_ref, vs_ref, acc_ref, m_ref, l_ref,
             vsum_ref):
    kb = pl.program_id(1)

    @pl.when(kb == 0)
    def _init():
        acc_ref[...] = jnp.zeros_like(acc_ref)
        m_ref[...] = jnp.full_like(m_ref, -3e38)
        l_ref[...] = jnp.zeros_like(l_ref)
        vsum_ref[...] = jnp.zeros_like(vsum_ref)

    lg = _mmb(qe_ref[0], k_ref[0], ((1,), (1,)))
    m_old = m_ref[...]
    m_new = jnp.maximum(m_old, jnp.max(lg, axis=1, keepdims=True))
    alpha = jnp.exp(m_old - m_new)
    p = jnp.exp(lg - m_new)
    l_ref[...] = l_ref[...] * alpha + jnp.sum(p, axis=1, keepdims=True)
    m_ref[...] = m_new
    acc_ref[...] = acc_ref[...] * alpha + _mmb(p, v_ref[0], ((1,), (0,)))
    vsum_ref[0:1, :] = vsum_ref[0:1, :] + jnp.sum(v_ref[0], axis=0,
                                                  keepdims=True)

    @pl.when(kb == (_L // _BK) - 1)
    def _fin():
        ov_ref[0] = acc_ref[...] / l_ref[...]
        vs_ref[0] = vsum_ref[...]


def _p4(qe, k, v):
    return pl.pallas_call(
        _p4_body,
        grid=(_B, _L // _BK),
        in_specs=[
            pl.BlockSpec((1, _NU, _D), lambda b, kb: (b, 0, 0)),
            pl.BlockSpec((1, _BK, _D), lambda b, kb: (b, kb, 0)),
            pl.BlockSpec((1, _BK, _D), lambda b, kb: (b, kb, 0)),
        ],
        out_specs=[
            pl.BlockSpec((1, _NU, _D), lambda b, kb: (b, 0, 0)),
            pl.BlockSpec((1, 8, _D), lambda b, kb: (b, 0, 0)),
        ],
        out_shape=[
            jax.ShapeDtypeStruct((_B, _NU, _D), _F32),
            jax.ShapeDtypeStruct((_B, 8, _D), _F32),
        ],
        scratch_shapes=[
            pltpu.VMEM((_NU, _D), _F32),
            pltpu.VMEM((_NU, 1), _F32),
            pltpu.VMEM((_NU, 1), _F32),
            pltpu.VMEM((8, _D), _F32),
        ],
    )(qe, k, v)


def _p6_body(ov_ref, vs_ref, wv_ref, bv_ref, wo_ref, bo_ref,
             delta_ref, base_ref):
    vmean_raw = vs_ref[0, 0:1, :] * (1.0 / _L)
    vmean = _mmb(vmean_raw, wv_ref[...], ((1,), (1,))) + bv_ref[...]
    vmb = vmean.astype(_BF16).astype(_F32)
    ov = _mmb(ov_ref[0], wv_ref[...], ((1,), (1,))) + bv_ref[...]
    parts = []
    for h in range(_H):
        c0 = h * _DK
        ot = ov[h * _U:(h + 1) * _U, c0:c0 + _DK]
        d_h = ot.astype(_BF16).astype(_F32) - vmb[0:1, c0:c0 + _DK]
        wo_h = wo_ref[:, c0:c0 + _DK].astype(_BF16).astype(_F32)
        parts.append(_mmf(d_h, wo_h, ((1,), (1,))))
    delta_ref[0] = jnp.concatenate(parts, axis=0)
    base_ref[0] = _mmb(vmean, wo_ref[...], ((1,), (1,))) + bo_ref[...]


def _p6(out_v, vs, Wv, bv2, Wo, bo2):
    return pl.pallas_call(
        _p6_body,
        grid=(_B,),
        in_specs=[
            pl.BlockSpec((1, _NU, _D), lambda b: (b, 0, 0)),
            pl.BlockSpec((1, 8, _D), lambda b: (b, 0, 0)),
            pl.BlockSpec((_D, _D), lambda b: (0, 0)),
            pl.BlockSpec((1, _D), lambda b: (0, 0)),
            pl.BlockSpec((_D, _D), lambda b: (0, 0)),
            pl.BlockSpec((1, _D), lambda b: (0, 0)),
        ],
        out_specs=[
            pl.BlockSpec((1, _NU, _D), lambda b: (b, 0, 0)),
            pl.BlockSpec((1, 1, _D), lambda b: (b, 0, 0)),
        ],
        out_shape=[
            jax.ShapeDtypeStruct((_B, _NU, _D), _F32),
            jax.ShapeDtypeStruct((_B, 1, _D), _F32),
        ],
    )(out_v, vs, Wv, bv2, Wo, bo2)


def _p7_body(base_ref, out_ref):
    out_ref[0] = jnp.broadcast_to(base_ref[0], (_BL, _D))


def _p7(base):
    return pl.pallas_call(
        _p7_body,
        grid=(_B, _L // _BL),
        in_specs=[pl.BlockSpec((1, 1, _D), lambda b, l: (b, 0, 0))],
        out_specs=pl.BlockSpec((1, _BL, _D), lambda b, l: (b, l, 0)),
        out_shape=jax.ShapeDtypeStruct((_B, _L, _D), _F32),
    )(base)


def kernel(q, k, v, Wq, bq, Wk, bk, Wv, bv, Wo, bo):
    bq2 = bq.reshape(1, _D)
    bk2 = bk.reshape(1, _D)
    bv2 = bv.reshape(1, _D)
    bo2 = bo.reshape(1, _D)
    bk_col = bk.reshape(_D, 1)

    ksamp_raw = k[:, _IDX_SAMPLE, :]
    ks2 = _p0(ksamp_raw, Wk, bk_col)
    m = _p1(q, Wq, bq2, ks2)

    _, top_idx = lax.top_k(m, _U)
    ti_flat = top_idx.reshape(_B, _NU)

    q_win = jnp.take_along_axis(q, ti_flat[..., None], axis=1)
    qe = _p3(q_win, Wq, bq2, Wk)
    out_v, vs = _p4(qe, k, v)
    delta, base = _p6(out_v, vs, Wv, bv2, Wo, bo2)

    out = _p7(base)
    b_idx = jnp.arange(_B)[:, None]
    out = out.at[b_idx, ti_flat].add(delta)
    return out


# speedup vs baseline: 1.1501x; 1.0909x over previous
import math

import numpy as np
import jax
import jax.numpy as jnp
from jax import lax
from jax.experimental import pallas as pl
from jax.experimental.pallas import tpu as pltpu

_B, _L, _D = 4, 8192, 768
_H, _DK, _U = 12, 64, 50
_NU = _H * _U
_BL = 512
_BK = 512

_F32 = jnp.float32
_BF16 = jnp.bfloat16

_IDX_SAMPLE = np.asarray([
    2357, 5665, 1885, 32, 4380, 7910, 4774, 7902, 3521, 7587, 3552, 5218,
    5424, 4781, 7884, 124, 1912, 5929, 235, 7940, 3146, 7584, 5586, 5739,
    6092, 5525, 4343, 2866, 7935, 7898, 6327, 7445, 7905, 2412, 3345, 1560,
    170, 4466, 7592, 7928, 2565, 5869, 5844, 1295, 2306, 1174, 5180, 412,
    3021, 7932], dtype=np.int32)


def _mmb(a, b, dims):
    return lax.dot_general(a.astype(_BF16), b.astype(_BF16),
                           (dims, ((), ())), preferred_element_type=_F32)


def _mmf(a, b, dims):
    return lax.dot_general(a, b, (dims, ((), ())),
                           preferred_element_type=_F32,
                           precision=lax.Precision.HIGHEST)


def _p0_body(ksr_ref, wk_ref, bkc_ref, ks2_ref):
    kst = _mmb(wk_ref[...], ksr_ref[0], ((1,), (1,))) + bkc_ref[...]
    row = lax.broadcasted_iota(jnp.int32, (_D, _DK), 0) // _DK
    col = lax.broadcasted_iota(jnp.int32, (_D, _DK), 1)
    pad = jnp.zeros((_D, _DK - _U), _F32)
    parts = []
    for h in range(_H):
        blk = jnp.concatenate([kst, pad], axis=1)
        parts.append(jnp.where((row == h) & (col < _U), blk, 0.0))
    ks2_ref[0] = jnp.concatenate(parts, axis=1)


def _p0(ksamp_raw, Wk, bk_col):
    return pl.pallas_call(
        _p0_body,
        grid=(_B,),
        in_specs=[
            pl.BlockSpec((1, _U, _D), lambda b: (b, 0, 0)),
            pl.BlockSpec((_D, _D), lambda b: (0, 0)),
            pl.BlockSpec((_D, 1), lambda b: (0, 0)),
        ],
        out_specs=pl.BlockSpec((1, _D, _D), lambda b: (b, 0, 0)),
        out_shape=jax.ShapeDtypeStruct((_B, _D, _D), _F32),
    )(ksamp_raw, Wk, bk_col)


def _p1_body(q_ref, wq_ref, bq_ref, ks2_ref, m_ref):
    qh = _mmb(q_ref[0], wq_ref[...], ((1,), (1,))) + bq_ref[...]
    s = _mmb(qh, ks2_ref[0], ((1,), (0,)))
    padb = jnp.where(
        lax.broadcasted_iota(jnp.int32, (1, _D), 1) % _DK < _U, 0.0, -3e38)
    smax = jnp.max((s + padb).reshape(_BL, _H, _DK), axis=-1)
    ssum = jnp.sum(s.reshape(_BL, _H, _DK), axis=-1)
    m_ref[0] = (smax - ssum / _U).T


def _p1(q, Wq, bq2, ks2):
    return pl.pallas_call(
        _p1_body,
        grid=(_B, _L // _BL),
        in_specs=[
            pl.BlockSpec((1, _BL, _D), lambda b, l: (b, l, 0)),
            pl.BlockSpec((_D, _D), lambda b, l: (0, 0)),
            pl.BlockSpec((1, _D), lambda b, l: (0, 0)),
            pl.BlockSpec((1, _D, _D), lambda b, l: (b, 0, 0)),
        ],
        out_specs=pl.BlockSpec((1, _H, _BL), lambda b, l: (b, 0, l)),
        out_shape=jax.ShapeDtypeStruct((_B, _H, _L), _F32),
    )(q, Wq, bq2, ks2)


def _p3_body(qw_ref, wq_ref, bq_ref, wk_ref, qe_ref):
    qt = _mmb(qw_ref[0], wq_ref[...], ((1,), (1,))) + bq_ref[...]
    rh = lax.broadcasted_iota(jnp.int32, (_NU, _D), 0) // _U
    ch = lax.broadcasted_iota(jnp.int32, (_NU, _D), 1) // _DK
    qtm = jnp.where(rh == ch, qt, 0.0)
    qe_ref[0] = _mmb(qtm, wk_ref[...], ((1,), (0,))) * 0.125


def _p3(q_win, Wq, bq2, Wk):
    return pl.pallas_call(
        _p3_body,
        grid=(_B,),
        in_specs=[
            pl.BlockSpec((1, _NU, _D), lambda b: (b, 0, 0)),
            pl.BlockSpec((_D, _D), lambda b: (0, 0)),
            pl.BlockSpec((1, _D), lambda b: (0, 0)),
            pl.BlockSpec((_D, _D), lambda b: (0, 0)),
        ],
        out_specs=pl.BlockSpec((1, _NU, _D), lambda b: (b, 0, 0)),
        out_shape=jax.ShapeDtypeStruct((_B, _NU, _D), _F32),
    )(q_win, Wq, bq2, Wk)


def _p4_body(qe_ref, k_ref, v_ref, ov_ref, vs_ref, acc_ref, m_ref, l_ref,
             vsum_ref):
    kb = pl.program_id(1)

    @pl.when(kb == 0)
    def _init():
        acc_ref[...] = jnp.zeros_like(acc_ref)
        m_ref[...] = jnp.full_like(m_ref, -3e38)
        l_ref[...] = jnp.zeros_like(l_ref)
        vsum_ref[...] = jnp.zeros_like(vsum_ref)

    lg = _mmb(qe_ref[0], k_ref[0], ((1,), (1,)))
    m_old = m_ref[...]
    m_new = jnp.maximum(m_old, jnp.max(lg, axis=1, keepdims=True))
    alpha = jnp.exp(m_old - m_new)
    p = jnp.exp(lg - m_new)
    l_ref[...] = l_ref[...] * alpha + jnp.sum(p, axis=1, keepdims=True)
    m_ref[...] = m_new
    acc_ref[...] = acc_ref[...] * alpha + _mmb(p, v_ref[0], ((1,), (0,)))
    vsum_ref[0:1, :] = vsum_ref[0:1, :] + jnp.sum(v_ref[0], axis=0,
                                                  keepdims=True)

    @pl.when(kb == (_L // _BK) - 1)
    def _fin():
        ov_ref[0] = acc_ref[...] / l_ref[...]
        vs_ref[0] = vsum_ref[...]


def _p4(qe, k, v):
    return pl.pallas_call(
        _p4_body,
        grid=(_B, _L // _BK),
        in_specs=[
            pl.BlockSpec((1, _NU, _D), lambda b, kb: (b, 0, 0)),
            pl.BlockSpec((1, _BK, _D), lambda b, kb: (b, kb, 0)),
            pl.BlockSpec((1, _BK, _D), lambda b, kb: (b, kb, 0)),
        ],
        out_specs=[
            pl.BlockSpec((1, _NU, _D), lambda b, kb: (b, 0, 0)),
            pl.BlockSpec((1, 8, _D), lambda b, kb: (b, 0, 0)),
        ],
        out_shape=[
            jax.ShapeDtypeStruct((_B, _NU, _D), _F32),
            jax.ShapeDtypeStruct((_B, 8, _D), _F32),
        ],
        scratch_shapes=[
            pltpu.VMEM((_NU, _D), _F32),
            pltpu.VMEM((_NU, 1), _F32),
            pltpu.VMEM((_NU, 1), _F32),
            pltpu.VMEM((8, _D), _F32),
        ],
    )(qe, k, v)


def _p6_body(ov_ref, vs_ref, wv_ref, bv_ref, wo_ref, bo_ref,
             delta_ref, base_ref):
    vmean_raw = vs_ref[0, 0:1, :] * (1.0 / _L)
    vmean = _mmb(vmean_raw, wv_ref[...], ((1,), (1,))) + bv_ref[...]
    vmb = vmean.astype(_BF16).astype(_F32)
    ov = _mmb(ov_ref[0], wv_ref[...], ((1,), (1,))) + bv_ref[...]
    parts = []
    for h in range(_H):
        c0 = h * _DK
        ot = ov[h * _U:(h + 1) * _U, c0:c0 + _DK]
        d_h = ot.astype(_BF16).astype(_F32) - vmb[0:1, c0:c0 + _DK]
        wo_h = wo_ref[:, c0:c0 + _DK].astype(_BF16).astype(_F32)
        parts.append(_mmf(d_h, wo_h, ((1,), (1,))))
    delta_ref[0] = jnp.concatenate(parts, axis=0)
    base_ref[0] = _mmb(vmean, wo_ref[...], ((1,), (1,))) + bo_ref[...]


def _p6(out_v, vs, Wv, bv2, Wo, bo2):
    return pl.pallas_call(
        _p6_body,
        grid=(_B,),
        in_specs=[
            pl.BlockSpec((1, _NU, _D), lambda b: (b, 0, 0)),
            pl.BlockSpec((1, 8, _D), lambda b: (b, 0, 0)),
            pl.BlockSpec((_D, _D), lambda b: (0, 0)),
            pl.BlockSpec((1, _D), lambda b: (0, 0)),
            pl.BlockSpec((_D, _D), lambda b: (0, 0)),
            pl.BlockSpec((1, _D), lambda b: (0, 0)),
        ],
        out_specs=[
            pl.BlockSpec((1, _NU, _D), lambda b: (b, 0, 0)),
            pl.BlockSpec((1, 1, _D), lambda b: (b, 0, 0)),
        ],
        out_shape=[
            jax.ShapeDtypeStruct((_B, _NU, _D), _F32),
            jax.ShapeDtypeStruct((_B, 1, _D), _F32),
        ],
    )(out_v, vs, Wv, bv2, Wo, bo2)


def _p7_body(base_ref, out_ref):
    out_ref[0] = jnp.broadcast_to(base_ref[0], (_BL, _D))


def _p7(base):
    return pl.pallas_call(
        _p7_body,
        grid=(_B, _L // _BL),
        in_specs=[pl.BlockSpec((1, 1, _D), lambda b, l: (b, 0, 0))],
        out_specs=pl.BlockSpec((1, _BL, _D), lambda b, l: (b, l, 0)),
        out_shape=jax.ShapeDtypeStruct((_B, _L, _D), _F32),
    )(base)


def kernel(q, k, v, Wq, bq, Wk, bk, Wv, bv, Wo, bo):
    bq2 = bq.reshape(1, _D)
    bk2 = bk.reshape(1, _D)
    bv2 = bv.reshape(1, _D)
    bo2 = bo.reshape(1, _D)
    bk_col = bk.reshape(_D, 1)

    ksamp_raw = k[:, _IDX_SAMPLE, :]
    ks2 = _p0(ksamp_raw, Wk, bk_col)
    m = _p1(q, Wq, bq2, ks2)

    _, top_idx = lax.top_k(m, _U)
    ti_flat = top_idx.reshape(_B, _NU)

    q_win = jnp.take_along_axis(q, ti_flat[..., None], axis=1)
    qe = _p3(q_win, Wq, bq2, Wk)
    out_v, vs = _p4(qe, k, v)
    delta, base = _p6(out_v, vs, Wv, bv2, Wo, bo2)

    out = _p7(base)
    b_idx = jnp.arange(_B)[:, None]
    out = out.at[b_idx, ti_flat].add(delta)
    return out


# speedup vs baseline: 1.2267x; 1.0666x over previous
import math

import numpy as np
import jax
import jax.numpy as jnp
from jax import lax
from jax.experimental import pallas as pl
from jax.experimental.pallas import tpu as pltpu

_B, _L, _D = 4, 8192, 768
_H, _DK, _U = 12, 64, 50
_NU = _H * _U
_BL = 1024
_BK = 1024

_F32 = jnp.float32
_BF16 = jnp.bfloat16

_IDX_SAMPLE = np.asarray([
    2357, 5665, 1885, 32, 4380, 7910, 4774, 7902, 3521, 7587, 3552, 5218,
    5424, 4781, 7884, 124, 1912, 5929, 235, 7940, 3146, 7584, 5586, 5739,
    6092, 5525, 4343, 2866, 7935, 7898, 6327, 7445, 7905, 2412, 3345, 1560,
    170, 4466, 7592, 7928, 2565, 5869, 5844, 1295, 2306, 1174, 5180, 412,
    3021, 7932], dtype=np.int32)


def _mmb(a, b, dims):
    return lax.dot_general(a.astype(_BF16), b.astype(_BF16),
                           (dims, ((), ())), preferred_element_type=_F32)


def _mmf(a, b, dims):
    return lax.dot_general(a, b, (dims, ((), ())),
                           preferred_element_type=_F32,
                           precision=lax.Precision.HIGHEST)


def _p0_body(ksr_ref, wk_ref, bkc_ref, ks2_ref):
    kst = _mmb(wk_ref[...], ksr_ref[0], ((1,), (1,))) + bkc_ref[...]
    row = lax.broadcasted_iota(jnp.int32, (_D, _DK), 0) // _DK
    col = lax.broadcasted_iota(jnp.int32, (_D, _DK), 1)
    pad = jnp.zeros((_D, _DK - _U), _F32)
    parts = []
    for h in range(_H):
        blk = jnp.concatenate([kst, pad], axis=1)
        parts.append(jnp.where((row == h) & (col < _U), blk, 0.0))
    ks2_ref[0] = jnp.concatenate(parts, axis=1)


def _p0(ksamp_raw, Wk, bk_col):
    return pl.pallas_call(
        _p0_body,
        grid=(_B,),
        in_specs=[
            pl.BlockSpec((1, _U, _D), lambda b: (b, 0, 0)),
            pl.BlockSpec((_D, _D), lambda b: (0, 0)),
            pl.BlockSpec((_D, 1), lambda b: (0, 0)),
        ],
        out_specs=pl.BlockSpec((1, _D, _D), lambda b: (b, 0, 0)),
        out_shape=jax.ShapeDtypeStruct((_B, _D, _D), _F32),
    )(ksamp_raw, Wk, bk_col)


def _p1_body(q_ref, wq_ref, bq_ref, ks2_ref, m_ref):
    qh = _mmb(q_ref[0], wq_ref[...], ((1,), (1,))) + bq_ref[...]
    s = _mmb(qh, ks2_ref[0], ((1,), (0,)))
    padb = jnp.where(
        lax.broadcasted_iota(jnp.int32, (1, _D), 1) % _DK < _U, 0.0, -3e38)
    smax = jnp.max((s + padb).reshape(_BL, _H, _DK), axis=-1)
    ssum = jnp.sum(s.reshape(_BL, _H, _DK), axis=-1)
    m_ref[0] = (smax - ssum / _U).T


def _p1(q, Wq, bq2, ks2):
    return pl.pallas_call(
        _p1_body,
        grid=(_B, _L // _BL),
        in_specs=[
            pl.BlockSpec((1, _BL, _D), lambda b, l: (b, l, 0)),
            pl.BlockSpec((_D, _D), lambda b, l: (0, 0)),
            pl.BlockSpec((1, _D), lambda b, l: (0, 0)),
            pl.BlockSpec((1, _D, _D), lambda b, l: (b, 0, 0)),
        ],
        out_specs=pl.BlockSpec((1, _H, _BL), lambda b, l: (b, 0, l)),
        out_shape=jax.ShapeDtypeStruct((_B, _H, _L), _F32),
    )(q, Wq, bq2, ks2)


def _p3_body(qw_ref, wq_ref, bq_ref, wk_ref, qe_ref):
    qt = _mmb(qw_ref[0], wq_ref[...], ((1,), (1,))) + bq_ref[...]
    rh = lax.broadcasted_iota(jnp.int32, (_NU, _D), 0) // _U
    ch = lax.broadcasted_iota(jnp.int32, (_NU, _D), 1) // _DK
    qtm = jnp.where(rh == ch, qt, 0.0)
    qe_ref[0] = _mmb(qtm, wk_ref[...], ((1,), (0,))) * 0.125


def _p3(q_win, Wq, bq2, Wk):
    return pl.pallas_call(
        _p3_body,
        grid=(_B,),
        in_specs=[
            pl.BlockSpec((1, _NU, _D), lambda b: (b, 0, 0)),
            pl.BlockSpec((_D, _D), lambda b: (0, 0)),
            pl.BlockSpec((1, _D), lambda b: (0, 0)),
            pl.BlockSpec((_D, _D), lambda b: (0, 0)),
        ],
        out_specs=pl.BlockSpec((1, _NU, _D), lambda b: (b, 0, 0)),
        out_shape=jax.ShapeDtypeStruct((_B, _NU, _D), _F32),
    )(q_win, Wq, bq2, Wk)


def _p4_body(qe_ref, k_ref, v_ref, ov_ref, vs_ref, acc_ref, m_ref, l_ref,
             vsum_ref):
    kb = pl.program_id(1)

    @pl.when(kb == 0)
    def _init():
        acc_ref[...] = jnp.zeros_like(acc_ref)
        m_ref[...] = jnp.full_like(m_ref, -3e38)
        l_ref[...] = jnp.zeros_like(l_ref)
        vsum_ref[...] = jnp.zeros_like(vsum_ref)

    lg = _mmb(qe_ref[0], k_ref[0], ((1,), (1,)))
    m_old = m_ref[...]
    m_new = jnp.maximum(m_old, jnp.max(lg, axis=1, keepdims=True))
    alpha = jnp.exp(m_old - m_new)
    p = jnp.exp(lg - m_new)
    l_ref[...] = l_ref[...] * alpha + jnp.sum(p, axis=1, keepdims=True)
    m_ref[...] = m_new
    acc_ref[...] = acc_ref[...] * alpha + _mmb(p, v_ref[0], ((1,), (0,)))
    vsum_ref[0:1, :] = vsum_ref[0:1, :] + jnp.sum(v_ref[0], axis=0,
                                                  keepdims=True)

    @pl.when(kb == (_L // _BK) - 1)
    def _fin():
        ov_ref[0] = acc_ref[...] / l_ref[...]
        vs_ref[0] = vsum_ref[...]


def _p4(qe, k, v):
    return pl.pallas_call(
        _p4_body,
        grid=(_B, _L // _BK),
        in_specs=[
            pl.BlockSpec((1, _NU, _D), lambda b, kb: (b, 0, 0)),
            pl.BlockSpec((1, _BK, _D), lambda b, kb: (b, kb, 0)),
            pl.BlockSpec((1, _BK, _D), lambda b, kb: (b, kb, 0)),
        ],
        out_specs=[
            pl.BlockSpec((1, _NU, _D), lambda b, kb: (b, 0, 0)),
            pl.BlockSpec((1, 8, _D), lambda b, kb: (b, 0, 0)),
        ],
        out_shape=[
            jax.ShapeDtypeStruct((_B, _NU, _D), _F32),
            jax.ShapeDtypeStruct((_B, 8, _D), _F32),
        ],
        scratch_shapes=[
            pltpu.VMEM((_NU, _D), _F32),
            pltpu.VMEM((_NU, 1), _F32),
            pltpu.VMEM((_NU, 1), _F32),
            pltpu.VMEM((8, _D), _F32),
        ],
    )(qe, k, v)


def _p6_body(ov_ref, vs_ref, wv_ref, bv_ref, wo_ref, bo_ref,
             delta_ref, base_ref):
    vmean_raw = vs_ref[0, 0:1, :] * (1.0 / _L)
    vmean = _mmb(vmean_raw, wv_ref[...], ((1,), (1,))) + bv_ref[...]
    vmb = vmean.astype(_BF16).astype(_F32)
    ov = _mmb(ov_ref[0], wv_ref[...], ((1,), (1,))) + bv_ref[...]
    parts = []
    for h in range(_H):
        c0 = h * _DK
        ot = ov[h * _U:(h + 1) * _U, c0:c0 + _DK]
        d_h = ot.astype(_BF16).astype(_F32) - vmb[0:1, c0:c0 + _DK]
        wo_h = wo_ref[:, c0:c0 + _DK].astype(_BF16).astype(_F32)
        parts.append(_mmf(d_h, wo_h, ((1,), (1,))))
    delta_ref[0] = jnp.concatenate(parts, axis=0)
    base_ref[0] = _mmb(vmean, wo_ref[...], ((1,), (1,))) + bo_ref[...]


def _p6(out_v, vs, Wv, bv2, Wo, bo2):
    return pl.pallas_call(
        _p6_body,
        grid=(_B,),
        in_specs=[
            pl.BlockSpec((1, _NU, _D), lambda b: (b, 0, 0)),
            pl.BlockSpec((1, 8, _D), lambda b: (b, 0, 0)),
            pl.BlockSpec((_D, _D), lambda b: (0, 0)),
            pl.BlockSpec((1, _D), lambda b: (0, 0)),
            pl.BlockSpec((_D, _D), lambda b: (0, 0)),
            pl.BlockSpec((1, _D), lambda b: (0, 0)),
        ],
        out_specs=[
            pl.BlockSpec((1, _NU, _D), lambda b: (b, 0, 0)),
            pl.BlockSpec((1, 1, _D), lambda b: (b, 0, 0)),
        ],
        out_shape=[
            jax.ShapeDtypeStruct((_B, _NU, _D), _F32),
            jax.ShapeDtypeStruct((_B, 1, _D), _F32),
        ],
    )(out_v, vs, Wv, bv2, Wo, bo2)


def _p7_body(base_ref, out_ref):
    out_ref[0] = jnp.broadcast_to(base_ref[0], (_BL, _D))


def _p7(base):
    return pl.pallas_call(
        _p7_body,
        grid=(_B, _L // _BL),
        in_specs=[pl.BlockSpec((1, 1, _D), lambda b, l: (b, 0, 0))],
        out_specs=pl.BlockSpec((1, _BL, _D), lambda b, l: (b, l, 0)),
        out_shape=jax.ShapeDtypeStruct((_B, _L, _D), _F32),
    )(base)


def kernel(q, k, v, Wq, bq, Wk, bk, Wv, bv, Wo, bo):
    bq2 = bq.reshape(1, _D)
    bk2 = bk.reshape(1, _D)
    bv2 = bv.reshape(1, _D)
    bo2 = bo.reshape(1, _D)
    bk_col = bk.reshape(_D, 1)

    ksamp_raw = k[:, _IDX_SAMPLE, :]
    ks2 = _p0(ksamp_raw, Wk, bk_col)
    m = _p1(q, Wq, bq2, ks2)

    _, top_idx = lax.top_k(m, _U)
    ti_flat = top_idx.reshape(_B, _NU)

    q_win = jnp.take_along_axis(q, ti_flat[..., None], axis=1)
    qe = _p3(q_win, Wq, bq2, Wk)
    out_v, vs = _p4(qe, k, v)
    delta, base = _p6(out_v, vs, Wv, bv2, Wo, bo2)

    out = _p7(base)
    b_idx = jnp.arange(_B)[:, None]
    out = out.at[b_idx, ti_flat].add(delta)
    return out


# speedup vs baseline: 1.3268x; 1.0817x over previous
import math

import numpy as np
import jax
import jax.numpy as jnp
from jax import lax
from jax.experimental import pallas as pl
from jax.experimental.pallas import tpu as pltpu

_B, _L, _D = 4, 8192, 768
_H, _DK, _U = 12, 64, 50
_NU = _H * _U
_BL = 1024
_BK = 1024

_F32 = jnp.float32
_BF16 = jnp.bfloat16

_IDX_SAMPLE = np.asarray([
    2357, 5665, 1885, 32, 4380, 7910, 4774, 7902, 3521, 7587, 3552, 5218,
    5424, 4781, 7884, 124, 1912, 5929, 235, 7940, 3146, 7584, 5586, 5739,
    6092, 5525, 4343, 2866, 7935, 7898, 6327, 7445, 7905, 2412, 3345, 1560,
    170, 4466, 7592, 7928, 2565, 5869, 5844, 1295, 2306, 1174, 5180, 412,
    3021, 7932], dtype=np.int32)


def _mmb(a, b, dims):
    return lax.dot_general(a.astype(_BF16), b.astype(_BF16),
                           (dims, ((), ())), preferred_element_type=_F32)


def _mmf(a, b, dims):
    return lax.dot_general(a, b, (dims, ((), ())),
                           preferred_element_type=_F32,
                           precision=lax.Precision.HIGHEST)


def _p0_body(ksr_ref, wk_ref, bkc_ref, ks2_ref):
    kst = _mmb(wk_ref[...], ksr_ref[0], ((1,), (1,))) + bkc_ref[...]
    row = lax.broadcasted_iota(jnp.int32, (_D, _DK), 0) // _DK
    col = lax.broadcasted_iota(jnp.int32, (_D, _DK), 1)
    pad = jnp.zeros((_D, _DK - _U), _F32)
    parts = []
    for h in range(_H):
        blk = jnp.concatenate([kst, pad], axis=1)
        parts.append(jnp.where((row == h) & (col < _U), blk, 0.0))
    ks2_ref[0] = jnp.concatenate(parts, axis=1)


def _p0(ksamp_raw, Wk, bk_col):
    return pl.pallas_call(
        _p0_body,
        grid=(_B,),
        in_specs=[
            pl.BlockSpec((1, _U, _D), lambda b: (b, 0, 0)),
            pl.BlockSpec((_D, _D), lambda b: (0, 0)),
            pl.BlockSpec((_D, 1), lambda b: (0, 0)),
        ],
        out_specs=pl.BlockSpec((1, _D, _D), lambda b: (b, 0, 0)),
        out_shape=jax.ShapeDtypeStruct((_B, _D, _D), _F32),
    )(ksamp_raw, Wk, bk_col)


def _p1_body(q_ref, wq_ref, bq_ref, ks2_ref, m_ref):
    qh = _mmb(q_ref[0], wq_ref[...], ((1,), (1,))) + bq_ref[...]
    s = _mmb(qh, ks2_ref[0], ((1,), (0,)))
    padb = jnp.where(
        lax.broadcasted_iota(jnp.int32, (1, _D), 1) % _DK < _U, 0.0, -3e38)
    smax = jnp.max((s + padb).reshape(_BL, _H, _DK), axis=-1)
    ssum = jnp.sum(s.reshape(_BL, _H, _DK), axis=-1)
    m_ref[0] = (smax - ssum / _U).T


def _p1(q, Wq, bq2, ks2):
    return pl.pallas_call(
        _p1_body,
        grid=(_B, _L // _BL),
        in_specs=[
            pl.BlockSpec((1, _BL, _D), lambda b, l: (b, l, 0)),
            pl.BlockSpec((_D, _D), lambda b, l: (0, 0)),
            pl.BlockSpec((1, _D), lambda b, l: (0, 0)),
            pl.BlockSpec((1, _D, _D), lambda b, l: (b, 0, 0)),
        ],
        out_specs=pl.BlockSpec((1, _H, _BL), lambda b, l: (b, 0, l)),
        out_shape=jax.ShapeDtypeStruct((_B, _H, _L), _F32),
    )(q, Wq, bq2, ks2)


def _p3_body(qw_ref, wq_ref, bq_ref, wk_ref, qe_ref):
    qt = _mmb(qw_ref[0], wq_ref[...], ((1,), (1,))) + bq_ref[...]
    rh = lax.broadcasted_iota(jnp.int32, (_NU, _D), 0) // _U
    ch = lax.broadcasted_iota(jnp.int32, (_NU, _D), 1) // _DK
    qtm = jnp.where(rh == ch, qt, 0.0)
    qe_ref[0] = _mmb(qtm, wk_ref[...], ((1,), (0,))) * 0.125


def _p3(q_win, Wq, bq2, Wk):
    return pl.pallas_call(
        _p3_body,
        grid=(_B,),
        in_specs=[
            pl.BlockSpec((1, _NU, _D), lambda b: (b, 0, 0)),
            pl.BlockSpec((_D, _D), lambda b: (0, 0)),
            pl.BlockSpec((1, _D), lambda b: (0, 0)),
            pl.BlockSpec((_D, _D), lambda b: (0, 0)),
        ],
        out_specs=pl.BlockSpec((1, _NU, _D), lambda b: (b, 0, 0)),
        out_shape=jax.ShapeDtypeStruct((_B, _NU, _D), _F32),
    )(q_win, Wq, bq2, Wk)


def _p4_body(qe_ref, k_ref, v_ref, ov_ref, vs_ref, acc_ref, m_ref, l_ref,
             vsum_ref):
    kb = pl.program_id(1)

    @pl.when(kb == 0)
    def _init():
        acc_ref[...] = jnp.zeros_like(acc_ref)
        m_ref[...] = jnp.full_like(m_ref, -3e38)
        l_ref[...] = jnp.zeros_like(l_ref)
        vsum_ref[...] = jnp.zeros_like(vsum_ref)

    lg = _mmb(qe_ref[0], k_ref[0], ((1,), (1,)))
    m_old = m_ref[...]
    m_new = jnp.maximum(m_old, jnp.max(lg, axis=1, keepdims=True))
    alpha = jnp.exp(m_old - m_new)
    p = jnp.exp(lg - m_new)
    l_ref[...] = l_ref[...] * alpha + jnp.sum(p, axis=1, keepdims=True)
    m_ref[...] = m_new
    acc_ref[...] = acc_ref[...] * alpha + _mmb(p, v_ref[0], ((1,), (0,)))
    vsum_ref[0:1, :] = vsum_ref[0:1, :] + jnp.sum(v_ref[0], axis=0,
                                                  keepdims=True)

    @pl.when(kb == (_L // _BK) - 1)
    def _fin():
        ov_ref[0] = acc_ref[...] / l_ref[...]
        vs_ref[0] = vsum_ref[...]


def _p4(qe, k, v):
    return pl.pallas_call(
        _p4_body,
        grid=(_B, _L // _BK),
        in_specs=[
            pl.BlockSpec((1, _NU, _D), lambda b, kb: (b, 0, 0)),
            pl.BlockSpec((1, _BK, _D), lambda b, kb: (b, kb, 0)),
            pl.BlockSpec((1, _BK, _D), lambda b, kb: (b, kb, 0)),
        ],
        out_specs=[
            pl.BlockSpec((1, _NU, _D), lambda b, kb: (b, 0, 0)),
            pl.BlockSpec((1, 8, _D), lambda b, kb: (b, 0, 0)),
        ],
        out_shape=[
            jax.ShapeDtypeStruct((_B, _NU, _D), _F32),
            jax.ShapeDtypeStruct((_B, 8, _D), _F32),
        ],
        scratch_shapes=[
            pltpu.VMEM((_NU, _D), _F32),
            pltpu.VMEM((_NU, 1), _F32),
            pltpu.VMEM((_NU, 1), _F32),
            pltpu.VMEM((8, _D), _F32),
        ],
    )(qe, k, v)


def _p6_body(ov_ref, vs_ref, wv_ref, bv_ref, wo_ref, bo_ref,
             delta_ref, base_ref):
    vmean_raw = vs_ref[0, 0:1, :] * (1.0 / _L)
    vmean = _mmb(vmean_raw, wv_ref[...], ((1,), (1,))) + bv_ref[...]
    vmb = vmean.astype(_BF16).astype(_F32)
    ov = _mmb(ov_ref[0], wv_ref[...], ((1,), (1,))) + bv_ref[...]
    parts = []
    for h in range(_H):
        c0 = h * _DK
        ot = ov[h * _U:(h + 1) * _U, c0:c0 + _DK]
        d_h = ot.astype(_BF16).astype(_F32) - vmb[0:1, c0:c0 + _DK]
        wo_h = wo_ref[:, c0:c0 + _DK].astype(_BF16).astype(_F32)
        parts.append(_mmf(d_h, wo_h, ((1,), (1,))))
    delta_ref[0] = jnp.concatenate(parts, axis=0)
    base_ref[0] = _mmb(vmean, wo_ref[...], ((1,), (1,))) + bo_ref[...]


def _p6(out_v, vs, Wv, bv2, Wo, bo2):
    return pl.pallas_call(
        _p6_body,
        grid=(_B,),
        in_specs=[
            pl.BlockSpec((1, _NU, _D), lambda b: (b, 0, 0)),
            pl.BlockSpec((1, 8, _D), lambda b: (b, 0, 0)),
            pl.BlockSpec((_D, _D), lambda b: (0, 0)),
            pl.BlockSpec((1, _D), lambda b: (0, 0)),
            pl.BlockSpec((_D, _D), lambda b: (0, 0)),
            pl.BlockSpec((1, _D), lambda b: (0, 0)),
        ],
        out_specs=[
            pl.BlockSpec((1, _NU, _D), lambda b: (b, 0, 0)),
            pl.BlockSpec((1, 1, _D), lambda b: (b, 0, 0)),
        ],
        out_shape=[
            jax.ShapeDtypeStruct((_B, _NU, _D), _F32),
            jax.ShapeDtypeStruct((_B, 1, _D), _F32),
        ],
    )(out_v, vs, Wv, bv2, Wo, bo2)


def _p7_body(ti_ref, perm_ref, offs_ref, base_ref, delta_ref, out_ref):
    b = pl.program_id(0)
    blk = pl.program_id(1)
    out_ref[0] = jnp.broadcast_to(base_ref[0], (_BL, _D))

    def _add(i, carry):
        r = ti_ref[b, i] - blk * _BL
        p = perm_ref[b, i]
        out_ref[0, pl.ds(r, 1), :] += delta_ref[0, pl.ds(p, 1), :]
        return carry

    lax.fori_loop(offs_ref[b, blk], offs_ref[b, blk + 1], _add, 0)


def _p7(ti_sorted, perm, offs, base, delta):
    return pl.pallas_call(
        _p7_body,
        grid_spec=pltpu.PrefetchScalarGridSpec(
            num_scalar_prefetch=3,
            grid=(_B, _L // _BL),
            in_specs=[
                pl.BlockSpec((1, 1, _D), lambda b, l, *_: (b, 0, 0)),
                pl.BlockSpec((1, _NU, _D), lambda b, l, *_: (b, 0, 0)),
            ],
            out_specs=pl.BlockSpec((1, _BL, _D), lambda b, l, *_: (b, l, 0)),
        ),
        out_shape=jax.ShapeDtypeStruct((_B, _L, _D), _F32),
    )(ti_sorted, perm, offs, base, delta)


def kernel(q, k, v, Wq, bq, Wk, bk, Wv, bv, Wo, bo):
    bq2 = bq.reshape(1, _D)
    bk2 = bk.reshape(1, _D)
    bv2 = bv.reshape(1, _D)
    bo2 = bo.reshape(1, _D)
    bk_col = bk.reshape(_D, 1)

    ksamp_raw = k[:, _IDX_SAMPLE, :]
    ks2 = _p0(ksamp_raw, Wk, bk_col)
    m = _p1(q, Wq, bq2, ks2)

    _, top_idx = lax.top_k(m, _U)
    ti_flat = top_idx.reshape(_B, _NU)

    q_win = jnp.take_along_axis(q, ti_flat[..., None], axis=1)
    qe = _p3(q_win, Wq, bq2, Wk)
    out_v, vs = _p4(qe, k, v)
    delta, base = _p6(out_v, vs, Wv, bv2, Wo, bo2)

    perm = jnp.argsort(ti_flat, axis=1).astype(jnp.int32)
    ti_sorted = jnp.take_along_axis(ti_flat, perm, axis=1)
    bounds = jnp.arange(0, _L + 1, _BL, dtype=jnp.int32)
    offs = jax.vmap(
        lambda t: jnp.searchsorted(t, bounds, side='left')
    )(ti_sorted).astype(jnp.int32)
    return _p7(ti_sorted, perm, offs, base, delta)
